# baseline (speedup 1.0000x reference)
import functools

import jax
import jax.numpy as jnp
from jax import lax
from jax.experimental import pallas as pl
from jax.experimental.pallas import tpu as pltpu
from jax.experimental.pallas import tpu_sc as plsc

_NC = 2
_NS = 16
_NW = _NC * _NS

_NB = 2


@functools.lru_cache(maxsize=None)
def _make_gather(V, D, B, S):
    bpw = B // _NW
    G = bpw
    M = G // _NB
    mesh = plsc.VectorSubcoreMesh(core_axis_name="c", subcore_axis_name="s")

    @functools.partial(
        pl.kernel,
        mesh=mesh,
        out_type=jax.ShapeDtypeStruct((B, S, D), jnp.float32),
        scratch_types=[
            pltpu.VMEM((bpw, S), jnp.int32),
            pltpu.VMEM((_NB, S, D), jnp.float32),
            pltpu.SemaphoreType.DMA,
            pltpu.SemaphoreType.DMA,
            pltpu.SemaphoreType.DMA,
            pltpu.SemaphoreType.DMA,
        ],
        compiler_params=pltpu.CompilerParams(use_tc_tiling_on_sc=False),
    )
    def k(table_hbm, idx_hbm, out_hbm, idx_v, rows_v, gs0, gs1, os0, os1):
        gsem = (gs0, gs1)
        osem = (os0, os1)
        wid = lax.axis_index("s") * _NC + lax.axis_index("c")
        base = wid * bpw
        pltpu.sync_copy(idx_hbm.at[pl.ds(base, bpw)], idx_v)

        def rows_slot(b):
            return rows_v.at[b]

        def fire_gather(cur, b):
            pltpu.async_copy(
                table_hbm.at[idx_v.at[cur]],
                rows_slot(b),
                gsem[b],
            )

        def wait_gather(b):
            pltpu.make_async_copy(
                out_hbm.at[0], rows_slot(b), gsem[b]
            ).wait()

        def fire_write(cur, b):
            pltpu.async_copy(
                rows_slot(b),
                out_hbm.at[base + cur],
                osem[b],
            )

        def wait_write(b):
            pltpu.make_async_copy(
                rows_slot(b), out_hbm.at[0], osem[b]
            ).wait()

        for b in range(_NB):
            fire_gather(b, b)

        def body(i, carry):
            for b in range(_NB):
                cur = i * _NB + b
                wait_gather(b)
                fire_write(cur, b)
                wait_write(b)
                fire_gather(cur + _NB, b)
            return carry

        lax.fori_loop(0, M - 1, body, 0)

        for b in range(_NB):
            wait_gather(b)
            fire_write((M - 1) * _NB + b, b)
        for b in range(_NB):
            wait_write(b)

    return k


def kernel(idx_texts, embed_table):
    B, S = idx_texts.shape
    V, D = embed_table.shape
    return _make_gather(V, D, B, S)(embed_table, idx_texts)
